# baseline (speedup 1.0000x reference)
import functools
import math

import jax
import jax.numpy as jnp
from jax import lax
from jax.experimental import pallas as pl
from jax.experimental.pallas import tpu as pltpu
from jax.experimental.pallas import tpu_sc as plsc

VOCAB = 1000000
D = 64
BATCH = 4096
SEQ = 200
SCALE = math.sqrt(D)

NC = 2
NS = 16
LANES = 16
NW = NC * NS
CPW = BATCH // NW
SB = 2
NCH = SEQ // SB
ROWS = CPW * SB


def _emb_body(x_hbm, tab_hbm, out_hbm, idx_v, rows_v, outb_v,
              gsem0, gsem1, osem0, osem1):
    wid = lax.axis_index("s") * NC + lax.axis_index("c")
    gsems = (gsem0, gsem1)
    osems = (osem0, osem1)
    iota = lax.iota(jnp.int32, LANES)

    pltpu.sync_copy(x_hbm.at[wid], idx_v)

    def gdescs(g, b):
        return [
            pltpu.make_async_copy(
                tab_hbm.at[idx_v.at[g, pl.ds(h * 128, 128)]],
                rows_v.at[b, pl.ds(h * 128, 128)],
                gsems[b])
            for h in range(ROWS // 128)
        ]

    def odesc(g, b):
        return pltpu.make_async_copy(
            outb_v.at[b], out_hbm.at[pl.ds(g * SB, SB), :, wid], osems[b])

    iota_sb = iota * SB

    def transpose_scale(b):
        def tbody(i, _):
            ds = i >> 6
            d = i & 63
            tr = d >> 3
            r = d & 7
            dv = jnp.full((LANES,), d, jnp.int32)
            for c16 in range(CPW // LANES):
                fvec = iota_sb + (c16 * LANES * SB + ds)
                v = plsc.load_gather(rows_v.at[b], [fvec, dv]) * SCALE
                outb_v[b, ds, tr, r, pl.ds(c16 * LANES, LANES)] = v
            return ()

        lax.fori_loop(0, SB * D, tbody, (), unroll=2)

    def substep(g, b, fire_next, wait_out):
        if fire_next:
            for dsc in gdescs(g + 1, 1 - b):
                dsc.start()
        for dsc in gdescs(g, b):
            dsc.wait()
        if wait_out:
            odesc(g - 2, b).wait()
        transpose_scale(b)
        odesc(g, b).start()

    for dsc in gdescs(0, 0):
        dsc.start()
    substep(0, 0, True, False)
    substep(1, 1, True, False)

    def pair(t, _):
        substep(2 * t, 0, True, True)
        substep(2 * t + 1, 1, True, True)
        return ()

    lax.fori_loop(1, NCH // 2 - 1, pair, ())

    substep(NCH - 2, 0, True, True)
    substep(NCH - 1, 1, False, True)
    odesc(NCH - 2, 0).wait()
    odesc(NCH - 1, 1).wait()


@jax.jit
def _emb_lookup(x3, emb_table):
    mesh = plsc.VectorSubcoreMesh(core_axis_name="c", subcore_axis_name="s")
    k = functools.partial(
        pl.kernel,
        out_type=jax.ShapeDtypeStruct((SEQ, D // 8, NW, 8, 128), jnp.float32),
        mesh=mesh,
        scratch_types=[
            pltpu.VMEM((NCH, ROWS), jnp.int32),
            pltpu.VMEM((2, ROWS, D), jnp.float32),
            pltpu.VMEM((2, SB, D // 8, 8, 128), jnp.float32),
            pltpu.SemaphoreType.DMA,
            pltpu.SemaphoreType.DMA,
            pltpu.SemaphoreType.DMA,
            pltpu.SemaphoreType.DMA,
        ],
        compiler_params=pltpu.CompilerParams(
            use_tc_tiling_on_sc=False, needs_layout_passes=False),
    )(_emb_body)
    return k(x3, emb_table)


def kernel(x, emb_table):
    x3 = (x.astype(jnp.int32)
          .reshape(NW, CPW, NCH, SB)
          .transpose(0, 2, 1, 3)
          .reshape(NW, NCH, ROWS))
    o5 = _emb_lookup(x3, emb_table)
    return o5.transpose(2, 4, 0, 1, 3).reshape(BATCH, SEQ, D)


# speedup vs baseline: 1.1334x; 1.1334x over previous
import functools
import math

import jax
import jax.numpy as jnp
from jax import lax
from jax.experimental import pallas as pl
from jax.experimental.pallas import tpu as pltpu
from jax.experimental.pallas import tpu_sc as plsc

VOCAB = 1000000
D = 64
BATCH = 4096
SEQ = 200
SCALE = math.sqrt(D)

NC = 2
NS = 16
LANES = 16
NW = NC * NS
CPW = BATCH // NW
SB = 2
NCH = SEQ // SB
ROWS = CPW * SB
OWORDS = SB * D * 128


def _emb_body(x_hbm, tab_hbm, out_hbm, idx_v, rows_v, outb_v,
              gsem0, gsem1, osem0, osem1):
    wid = lax.axis_index("s") * NC + lax.axis_index("c")
    gsems = (gsem0, gsem1)
    osems = (osem0, osem1)
    iota = lax.iota(jnp.int32, LANES)
    perm = (iota >> 3) * 1024 + (iota & 7) * 128
    perms = [perm + (d16 * 2 * 1024) for d16 in range(D // LANES)]

    pltpu.sync_copy(x_hbm.at[wid], idx_v)

    def gdescs(g, b):
        return [
            pltpu.make_async_copy(
                tab_hbm.at[idx_v.at[g, pl.ds(h * 128, 128)]],
                rows_v.at[b, pl.ds(h * 128, 128)],
                gsems[b])
            for h in range(ROWS // 128)
        ]

    def odescs(g, b):
        descs = []
        for ds in range(SB):
            s = g * SB + ds
            for tr in range(D // 8):
                row = (s * (D // 8) + tr) * NW + wid
                descs.append(pltpu.make_async_copy(
                    outb_v.at[b, pl.ds((ds * (D // 8) + tr) * 1024, 1024)],
                    out_hbm.at[row],
                    osems[b]))
        return descs

    def transpose_scale(b):
        def tbody(row, _):
            ds = row & 1
            c = row >> 1
            basev = jnp.full((LANES,), ds * (D * 128) + c, jnp.int32)
            for d16 in range(D // LANES):
                v = rows_v[b, row, pl.ds(d16 * LANES, LANES)] * SCALE
                plsc.store_scatter(outb_v.at[b], [perms[d16] + basev], v)
            return ()

        lax.fori_loop(0, ROWS, tbody, (), unroll=8)

    def substep(g, b, fire_next, wait_out):
        if fire_next:
            for dsc in gdescs(g + 1, 1 - b):
                dsc.start()
        for dsc in gdescs(g, b):
            dsc.wait()
        if wait_out:
            for dsc in odescs(g - 2, b):
                dsc.wait()
        transpose_scale(b)
        for dsc in odescs(g, b):
            dsc.start()

    for dsc in gdescs(0, 0):
        dsc.start()
    substep(0, 0, True, False)
    substep(1, 1, True, False)

    def pair(t, _):
        substep(2 * t, 0, True, True)
        substep(2 * t + 1, 1, True, True)
        return ()

    lax.fori_loop(1, NCH // 2 - 1, pair, ())

    substep(NCH - 2, 0, True, True)
    substep(NCH - 1, 1, False, True)
    for dsc in odescs(NCH - 2, 0):
        dsc.wait()
    for dsc in odescs(NCH - 1, 1):
        dsc.wait()


@jax.jit
def _emb_lookup(x3, emb_table):
    mesh = plsc.VectorSubcoreMesh(core_axis_name="c", subcore_axis_name="s")
    k = functools.partial(
        pl.kernel,
        out_type=jax.ShapeDtypeStruct((SEQ * (D // 8) * NW, 1024), jnp.float32),
        mesh=mesh,
        scratch_types=[
            pltpu.VMEM((NCH, ROWS), jnp.int32),
            pltpu.VMEM((2, ROWS, D), jnp.float32),
            pltpu.VMEM((2, OWORDS), jnp.float32),
            pltpu.SemaphoreType.DMA,
            pltpu.SemaphoreType.DMA,
            pltpu.SemaphoreType.DMA,
            pltpu.SemaphoreType.DMA,
        ],
        compiler_params=pltpu.CompilerParams(
            use_tc_tiling_on_sc=False, needs_layout_passes=False),
    )(_emb_body)
    return k(x3, emb_table)


def kernel(x, emb_table):
    x3 = (x.astype(jnp.int32)
          .reshape(NW, CPW, NCH, SB)
          .transpose(0, 2, 1, 3)
          .reshape(NW, NCH, ROWS))
    o = _emb_lookup(x3, emb_table)
    return (o.reshape(SEQ, D // 8, NW, 8, 128)
            .transpose(2, 4, 0, 1, 3)
            .reshape(BATCH, SEQ, D))


# speedup vs baseline: 1.4564x; 1.2850x over previous
import functools
import math

import jax
import jax.numpy as jnp
from jax import lax
from jax.experimental import pallas as pl
from jax.experimental.pallas import tpu as pltpu
from jax.experimental.pallas import tpu_sc as plsc

VOCAB = 1000000
D = 64
BATCH = 4096
SEQ = 200
SCALE = math.sqrt(D)

NC = 2
NS = 16
LANES = 16
NW = NC * NS
CPW = BATCH // NW
SB = 2
NCH = SEQ // SB
ROWS = CPW * SB
OWORDS = SB * D * 128


def _emb_body(x_hbm, tab_hbm, out_hbm, idx_v, rows_v, outb_v,
              gsem0, gsem1, osem0, osem1):
    wid = lax.axis_index("s") * NC + lax.axis_index("c")
    gsems = (gsem0, gsem1)
    osems = (osem0, osem1)
    iota = lax.iota(jnp.int32, LANES)
    perm = (iota >> 3) * 1024 + (iota & 7) * 128
    perms = [perm + (d16 * 2 * 1024) for d16 in range(D // LANES)]

    pltpu.sync_copy(x_hbm.at[wid], idx_v)

    def gdescs(g, b):
        return [
            pltpu.make_async_copy(
                tab_hbm.at[idx_v.at[g, pl.ds(h * 128, 128)]],
                rows_v.at[b, pl.ds(h * 128, 128)],
                gsems[b])
            for h in range(ROWS // 128)
        ]

    def odescs(g, b):
        descs = []
        for ds in range(SB):
            s = g * SB + ds
            for tr in range(D // 8):
                row = (s * (D // 8) + tr) * NW + wid
                descs.append(pltpu.make_async_copy(
                    outb_v.at[b, pl.ds((ds * (D // 8) + tr) * 1024, 1024)],
                    out_hbm.at[row],
                    osems[b]))
        return descs

    def transpose_scale(b):
        @plsc.parallel_loop(0, ROWS, unroll=8)
        def tbody(row):
            ds = row & 1
            c = row >> 1
            basev = jnp.full((LANES,), ds * (D * 128) + c, jnp.int32)
            for d16 in range(D // LANES):
                v = rows_v[b, row, pl.ds(d16 * LANES, LANES)] * SCALE
                plsc.store_scatter(outb_v.at[b], [perms[d16] + basev], v)

    def substep(g, b, fire_next, wait_out):
        if fire_next:
            for dsc in gdescs(g + 1, 1 - b):
                dsc.start()
        for dsc in gdescs(g, b):
            dsc.wait()
        if wait_out:
            for dsc in odescs(g - 2, b):
                dsc.wait()
        transpose_scale(b)
        for dsc in odescs(g, b):
            dsc.start()

    for dsc in gdescs(0, 0):
        dsc.start()
    substep(0, 0, True, False)
    substep(1, 1, True, False)

    def pair(t, _):
        substep(2 * t, 0, True, True)
        substep(2 * t + 1, 1, True, True)
        return ()

    lax.fori_loop(1, NCH // 2 - 1, pair, ())

    substep(NCH - 2, 0, True, True)
    substep(NCH - 1, 1, False, True)
    for dsc in odescs(NCH - 2, 0):
        dsc.wait()
    for dsc in odescs(NCH - 1, 1):
        dsc.wait()


@jax.jit
def _emb_lookup(x3, emb_table):
    mesh = plsc.VectorSubcoreMesh(core_axis_name="c", subcore_axis_name="s")
    k = functools.partial(
        pl.kernel,
        out_type=jax.ShapeDtypeStruct((SEQ * (D // 8) * NW, 1024), jnp.float32),
        mesh=mesh,
        scratch_types=[
            pltpu.VMEM((NCH, ROWS), jnp.int32),
            pltpu.VMEM((2, ROWS, D), jnp.float32),
            pltpu.VMEM((2, OWORDS), jnp.float32),
            pltpu.SemaphoreType.DMA,
            pltpu.SemaphoreType.DMA,
            pltpu.SemaphoreType.DMA,
            pltpu.SemaphoreType.DMA,
        ],
        compiler_params=pltpu.CompilerParams(
            use_tc_tiling_on_sc=False, needs_layout_passes=False),
    )(_emb_body)
    return k(x3, emb_table)


def kernel(x, emb_table):
    x3 = (x.astype(jnp.int32)
          .reshape(NW, CPW, NCH, SB)
          .transpose(0, 2, 1, 3)
          .reshape(NW, NCH, ROWS))
    o = _emb_lookup(x3, emb_table)
    return (o.reshape(SEQ, D // 8, NW, 8, 128)
            .transpose(2, 4, 0, 1, 3)
            .reshape(BATCH, SEQ, D))


# speedup vs baseline: 2.0915x; 1.4361x over previous
import functools
import math

import jax
import jax.numpy as jnp
from jax import lax
from jax.experimental import pallas as pl
from jax.experimental.pallas import tpu as pltpu
from jax.experimental.pallas import tpu_sc as plsc

VOCAB = 1000000
D = 64
BATCH = 4096
SEQ = 200
SCALE = math.sqrt(D)

NC = 2
NS = 16
LANES = 16
NW = NC * NS

TOTAL = BATCH * SEQ
PER_W = TOTAL // NW
CHUNK = 128
KPG = 4
G = CHUNK * KPG
NG = PER_W // G
NCHUNK = PER_W // CHUNK


def _emb_body(x_hbm, tab_hbm, out_hbm, idx_v, rows_v, gsem0, gsem1):
    wid = lax.axis_index("s") * NC + lax.axis_index("c")
    gsems = (gsem0, gsem1)
    pltpu.sync_copy(x_hbm.at[wid], idx_v)

    def gather_desc(g, b, j):
        return pltpu.make_async_copy(
            tab_hbm.at[idx_v.at[g * KPG + j]],
            rows_v.at[b, pl.ds(j * CHUNK, CHUNK)],
            gsems[b],
        )

    def fire(g, b):
        for j in range(KPG):
            gather_desc(g, b, j).start()

    def process(g, b, do_fire):
        for j in range(KPG):
            gather_desc(g, b, j).wait()

        def mrow(r, _):
            for q in range(D // LANES):
                sl = pl.ds(q * LANES, LANES)
                rows_v[b, r, sl] = rows_v[b, r, sl] * SCALE
            return ()

        lax.fori_loop(0, G, mrow, (), unroll=8)
        pltpu.sync_copy(
            rows_v.at[b],
            out_hbm.at[pl.ds(wid * PER_W + g * G, G), pl.ds(0, D)])
        if do_fire:
            fire(g + 2, b)

    fire(0, 0)
    fire(1, 1)

    def step(t, _):
        process(2 * t, 0, True)
        process(2 * t + 1, 1, True)
        return ()

    lax.fori_loop(0, NG // 2 - 1, step, ())
    process(NG - 2, 0, False)
    process(NG - 1, 1, False)


@jax.jit
def _emb_lookup(x2d, emb_table):
    mesh = plsc.VectorSubcoreMesh(core_axis_name="c", subcore_axis_name="s")
    k = functools.partial(
        pl.kernel,
        out_type=jax.ShapeDtypeStruct((TOTAL, 128), jnp.float32),
        mesh=mesh,
        scratch_types=[
            pltpu.VMEM((NCHUNK, CHUNK), jnp.int32),
            pltpu.VMEM((2, G, D), jnp.float32),
            pltpu.SemaphoreType.DMA,
            pltpu.SemaphoreType.DMA,
        ],
        compiler_params=pltpu.CompilerParams(use_tc_tiling_on_sc=False),
    )(_emb_body)
    return k(x2d, emb_table)


def kernel(x, emb_table):
    x2d = x.astype(jnp.int32).reshape(NW, NCHUNK, CHUNK)
    out = _emb_lookup(x2d, emb_table)
    return out[:, :D].reshape(BATCH, SEQ, D)
